# baseline (speedup 1.0000x reference)
import functools

import jax
import jax.numpy as jnp
from jax.experimental import pallas as pl
from jax.experimental.pallas import tpu as pltpu

B = 256
N = 64
ENC = 128
DEC = 64
DEC2 = 32
RB = 8
ROWS = RB * N
STEPS = DEC // RB


def _goggle_kernel(x_ref, we_ref, be_ref, wmu_ref, bmu_ref, wlv_ref, blv_ref,
                   g_ref, w0f_ref, ccf_ref, w1m_ref, b1e_ref, w2m_ref, b2_ref,
                   it_ref, eps_ref,
                   xhat_ref, adj_ref, mu_ref, lv_ref,
                   bz_ref, acc_ref, adjexp_ref):
    k = pl.program_id(0)

    @pl.when(k == 0)
    def _prologue():
        h = jax.nn.relu(jnp.dot(x_ref[...], we_ref[...],
                                preferred_element_type=jnp.float32) + be_ref[...])
        mu = jnp.dot(h, wmu_ref[...], preferred_element_type=jnp.float32) + bmu_ref[...]
        lv = jnp.dot(h, wlv_ref[...], preferred_element_type=jnp.float32) + blv_ref[...]
        mu_ref[...] = mu
        lv_ref[...] = lv
        z = mu + eps_ref[...] * jnp.exp(0.5 * lv)

        r_id = jax.lax.broadcasted_iota(jnp.int32, (N, N), 0)
        c_id = jax.lax.broadcasted_iota(jnp.int32, (N, N), 1)
        eye = (r_id == c_id).astype(jnp.float32)
        adj = jax.nn.sigmoid(g_ref[...]) * (1.0 - eye) + eye
        adj = jnp.where(jnp.logical_and(it_ref[0, 0] > 50.0, adj <= 0.1), 0.0, adj)
        adj_ref[...] = adj
        adjexp_ref[...] = jnp.concatenate([adj] * DEC2, axis=1)

        zexp = jnp.concatenate([z] * DEC, axis=1)
        bz_ref[...] = jnp.tanh(zexp * w0f_ref[...] + ccf_ref[...])
        acc_ref[...] = jnp.zeros((B, DEC2 * N), dtype=jnp.float32)

    scale = jnp.concatenate([adjexp_ref[...]] * RB, axis=0)
    lhs = bz_ref[:, pl.ds(k * ROWS, ROWS)]
    acc_ref[...] += jnp.dot(lhs, w1m_ref[...] * scale,
                            preferred_element_type=jnp.float32)

    @pl.when(k == STEPS - 1)
    def _epilogue():
        h1 = jax.nn.relu(acc_ref[...] + b1e_ref[...])
        s2 = jnp.concatenate([adj_ref[...]] * DEC2, axis=0)
        xhat_ref[...] = jnp.dot(h1, w2m_ref[...] * s2,
                                preferred_element_type=jnp.float32) + b2_ref[...]


@functools.partial(jax.jit, static_argnames=())
def kernel(x, We, be, Wmu, bmu, Wlv, blv, G, Wemb, bemb, W1, b1, W2, b2, iter):
    f32 = jnp.float32
    w0f = Wemb[:, 0, :].T.reshape(1, DEC * N)
    ccf = (Wemb[jnp.arange(N), jnp.arange(N) + 1, :] + bemb).T.reshape(1, DEC * N)
    w1m = W1.transpose(2, 0, 3, 1).reshape(DEC * N, DEC2 * N)
    w2m = W2[:, :, :, 0].transpose(2, 0, 1).reshape(DEC2 * N, N)
    b1e = jnp.repeat(b1, N).reshape(1, DEC2 * N)
    eps = jax.random.normal(jax.random.key(42), (B, N), dtype=f32)
    it = jnp.asarray(iter, dtype=f32).reshape(1, 1)

    grid = (STEPS,)
    resident = lambda s: pl.BlockSpec(s, lambda k: (0,) * len(s))
    out = pl.pallas_call(
        _goggle_kernel,
        grid=grid,
        in_specs=[
            resident((B, N)),
            resident((N, ENC)),
            resident((1, ENC)),
            resident((ENC, N)),
            resident((1, N)),
            resident((ENC, N)),
            resident((1, N)),
            resident((N, N)),
            resident((1, DEC * N)),
            resident((1, DEC * N)),
            pl.BlockSpec((ROWS, DEC2 * N), lambda k: (k, 0)),
            resident((1, DEC2 * N)),
            resident((DEC2 * N, N)),
            resident((1, 1)),
            resident((1, 1)),
            resident((B, N)),
        ],
        out_specs=(
            resident((B, N)),
            resident((N, N)),
            resident((B, N)),
            resident((B, N)),
        ),
        out_shape=(
            jax.ShapeDtypeStruct((B, N), f32),
            jax.ShapeDtypeStruct((N, N), f32),
            jax.ShapeDtypeStruct((B, N), f32),
            jax.ShapeDtypeStruct((B, N), f32),
        ),
        scratch_shapes=[
            pltpu.VMEM((B, DEC * N), f32),
            pltpu.VMEM((B, DEC2 * N), f32),
            pltpu.VMEM((N, DEC2 * N), f32),
        ],
        compiler_params=pltpu.CompilerParams(
            dimension_semantics=("arbitrary",),
        ),
    )(x, We, be.reshape(1, ENC), Wmu, bmu.reshape(1, N), Wlv, blv.reshape(1, N),
      G, w0f, ccf, w1m, b1e, w2m, b2.reshape(1, 1), it, eps)
    return out
